# baseline (speedup 1.0000x reference)
import functools

import jax
import jax.numpy as jnp
from jax import lax
from jax.experimental import pallas as pl
from jax.experimental.pallas import tpu as pltpu
from jax.experimental.pallas import tpu_sc as plsc

U_NUM = 25000
I_NUM = 25000
N = U_NUM + I_NUM
E = 800000
D = 64
H = 32
NLAYERS = 3

NC = 2
NS = 16
IW = 128
RW = 6400
EP = RW * IW
ROWS_PT = RW // NS
K = 1
NCHUNK = ROWS_PT // K
NP = 51200
RPT = NP // NS
ZR = 160


def _sc_body(egoT, eidx, l1, l2, mout,
             acc, ebuf0, ebuf1, ebuf2, ebuf3, rows0, rows1, rows2, rows3, zer,
             gsem0, gsem1, gsem2, gsem3, ssem0, ssem1, ssem2, ssem3,
             isem0, isem1, isem2, isem3):
    c = lax.axis_index("c")
    s = lax.axis_index("s")
    ebufs = [ebuf0, ebuf1, ebuf2, ebuf3]
    isems = [isem0, isem1, isem2, isem3]
    rowss = [rows0, rows1, rows2, rows3]
    gsems = [gsem0, gsem1, gsem2, gsem3]
    ssems = [ssem0, ssem1, ssem2, ssem3]
    coff = c * NP

    zf = jnp.zeros((16,), jnp.float32)

    def zfill(i, _):
        zer[i, pl.ds(0, 16)] = zf
        zer[i, pl.ds(16, 16)] = zf
        return 0

    lax.fori_loop(0, ZR, zfill, 0, unroll=4)

    base_r = s * RPT
    erow0 = s * ROWS_PT

    def idx_slice(kk):
        rn = jnp.minimum(kk, NCHUNK - 1)
        return eidx.at[pl.ds(erow0 + rn * K, K)]

    def fire_idx(kk, slot):
        pltpu.async_copy(idx_slice(kk), ebufs[slot], isems[slot])

    def wait_idx(kk, slot):
        pltpu.make_async_copy(idx_slice(kk), ebufs[slot], isems[slot]).wait()

    def adjust_src(slot):
        for g in range(IW // 16):
            ebufs[slot][0, 1, pl.ds(g * 16, 16)] = \
                ebufs[slot][0, 1, pl.ds(g * 16, 16)] + coff

    def one_layer(table, out, last=False):
        zcps = [pltpu.async_copy(zer, acc.at[pl.ds(base_r + z * ZR, ZR)],
                                 gsem0)
                for z in range(RPT // ZR)]
        for zc in zcps:
            zc.wait()
        plsc.subcore_barrier()

        def fire_g(m):
            pltpu.async_copy(table.at[ebufs[m].at[0, 1]],
                             rowss[m].at[0], gsems[m])

        def wait_g(m):
            pltpu.make_async_copy(table.at[ebufs[m].at[0, 1]],
                                  rowss[m].at[0], gsems[m]).wait()

        def fire_s(m):
            pltpu.async_copy(rowss[m].at[0], acc.at[ebufs[m].at[0, 0]],
                             ssems[m], add=True)

        def wait_s(m):
            pltpu.make_async_copy(rowss[m].at[0],
                                  acc.at[ebufs[m].at[0, 0]],
                                  ssems[m]).wait()

        def mul(m):
            def wmul(g, _):
                w16 = lax.bitcast_convert_type(
                    ebufs[m][0, 2, pl.ds(g * 16, 16)], jnp.float32)
                for i in range(16):
                    w = lax.gather(
                        w16, jnp.full((16, 1), i, jnp.int32),
                        dimension_numbers=lax.GatherDimensionNumbers(
                            offset_dims=(), collapsed_slice_dims=(0,),
                            start_index_map=(0,)),
                        slice_sizes=(1,),
                        mode=lax.GatherScatterMode.PROMISE_IN_BOUNDS)
                    r = g * 16 + i
                    rb = rowss[m]
                    rb[0, r, pl.ds(0, 16)] = rb[0, r, pl.ds(0, 16)] * w
                    rb[0, r, pl.ds(16, 16)] = rb[0, r, pl.ds(16, 16)] * w
                return 0

            lax.fori_loop(0, IW // 16, wmul, 0)

        fire_idx(0, 0)
        fire_idx(1, 1)
        wait_idx(0, 0)
        adjust_src(0)
        fire_g(0)

        @pl.loop(0, NCHUNK, step=4)
        def grp(k):
            for b in range(4):
                kk = k + b
                bn, bp = (b + 1) % 4, (b + 2) % 4

                def steady(b=b, bn=bn, bp=bp, kk=kk):
                    wait_s(bp)
                    fire_idx(kk + 2, bp)
                if b < 2:
                    pl.when(k > 0)(steady)
                    if b == 1:
                        @pl.when(k == 0)
                        def _(bp=bp, kk=kk):
                            fire_idx(kk + 2, bp)
                    else:
                        @pl.when(k == 0)
                        def _(bp=bp, kk=kk):
                            fire_idx(kk + 2, bp)
                else:
                    steady()
                wait_idx(kk + 1, bn)
                adjust_src(bn)
                fire_g(bn)
                wait_g(b)
                mul(b)
                fire_s(b)

        wait_s(2)
        wait_s(3)
        wait_g(0)
        wait_idx(NCHUNK + 1, 1)

        plsc.subcore_barrier()
        if not last:
            wcps = [pltpu.async_copy(
                        acc.at[pl.ds(base_r + z * ZR, ZR)],
                        out.at[pl.ds(c * NP + base_r + z * ZR, ZR)], gsem0)
                    for z in range(RPT // ZR)]
            for wc in wcps:
                wc.wait()
        else:
            MQ = 128
            third = jnp.float32(1.0 / NLAYERS)

            @pl.loop(0, RPT // MQ)
            def mq(qi):
                row = base_r + qi * MQ
                cp1 = pltpu.async_copy(l1.at[pl.ds(c * NP + row, MQ)],
                                       rows0.at[0], gsem0)
                cp2 = pltpu.async_copy(l2.at[pl.ds(c * NP + row, MQ)],
                                       rows1.at[0], gsem1)
                cp3 = pltpu.async_copy(acc.at[pl.ds(row, MQ)],
                                       rows2.at[0], gsem2)
                cp1.wait()
                cp2.wait()
                cp3.wait()

                def mbody(i, _):
                    for hh in (0, 16):
                        v = (rows0[0, i, pl.ds(hh, 16)]
                             + rows1[0, i, pl.ds(hh, 16)]
                             + rows2[0, i, pl.ds(hh, 16)])
                        rows3[0, i, pl.ds(hh, 16)] = v * third
                    return 0

                lax.fori_loop(0, MQ, mbody, 0, unroll=4)
                pltpu.sync_copy(rows3.at[0], out.at[pl.ds(c * NP + row, MQ)])
        plsc.subcore_barrier()

    one_layer(egoT, l1)
    one_layer(l1, l2)
    one_layer(l2, mout, last=True)


def _propagate(egoT, eidx):
    mesh = plsc.VectorSubcoreMesh(core_axis_name="c", subcore_axis_name="s")
    sds = jax.ShapeDtypeStruct((2 * NP, H), jnp.float32)
    return pl.kernel(
        _sc_body,
        out_type=(sds, sds, sds),
        mesh=mesh,
        scratch_types=[
            pltpu.VMEM_SHARED((NP, H), jnp.float32),
        ] + [pltpu.VMEM((K, 3, IW), jnp.int32)] * 4
          + [pltpu.VMEM((K, IW, H), jnp.float32)] * 4
          + [pltpu.VMEM((ZR, H), jnp.float32)]
          + [pltpu.SemaphoreType.DMA] * 12,
        compiler_params=pltpu.CompilerParams(use_tc_tiling_on_sc=False),
    )(egoT, eidx)


def _assemble_body(a, b, o):
    o[:, :H] = a[...]
    o[:, H:] = b[...]


def _assemble(mout):
    BR = 400
    return pl.pallas_call(
        _assemble_body,
        grid=(N // BR,),
        in_specs=[pl.BlockSpec((BR, H), lambda i: (i, 0)),
                  pl.BlockSpec((BR, H), lambda i: (i + NP // BR, 0))],
        out_specs=pl.BlockSpec((BR, D), lambda i: (i, 0)),
        out_shape=jax.ShapeDtypeStruct((N, D), jnp.float32),
    )(mout, mout)


@jax.jit
def kernel(user_emb, item_emb, user_prototypes, item_prototypes,
           edge_index, edge_weight):
    zpadT = jnp.zeros((NP - N, H), jnp.float32)
    egoT = jnp.concatenate(
        [user_emb[:, :H], item_emb[:, :H], zpadT,
         user_emb[:, H:], item_emb[:, H:], zpadT], axis=0)

    pad = EP - E
    dst_p = jnp.concatenate([edge_index[0], jnp.zeros((pad,), jnp.int32)])
    src_p = jnp.concatenate([edge_index[1], jnp.zeros((pad,), jnp.int32)])
    w_p = jnp.concatenate([edge_weight, jnp.zeros((pad,), jnp.float32)])
    wbits = lax.bitcast_convert_type(w_p, jnp.int32)
    eidx = jnp.concatenate(
        [dst_p.reshape(RW, 1, IW), src_p.reshape(RW, 1, IW),
         wbits.reshape(RW, 1, IW)], axis=1)

    l1, l2, mout = _propagate(egoT, eidx)

    all_emb = _assemble(mout)
    return (all_emb[:U_NUM], all_emb[U_NUM:],
            user_prototypes, item_prototypes)


# speedup vs baseline: 1.0432x; 1.0432x over previous
import functools

import jax
import jax.numpy as jnp
from jax import lax
from jax.experimental import pallas as pl
from jax.experimental.pallas import tpu as pltpu
from jax.experimental.pallas import tpu_sc as plsc

U_NUM = 25000
I_NUM = 25000
N = U_NUM + I_NUM
E = 800000
D = 64
H = 32
NLAYERS = 3

NC = 2
NS = 16
IW = 128
RW = 6400
EP = RW * IW
ROWS_PT = RW // NS
K = 1
NCHUNK = ROWS_PT // K
NP = 51200
RPT = NP // NS
ZR = 160


def _sc_body(egoT, eidx, l1, l2, mout,
             acc, ebuf0, ebuf1, ebuf2, ebuf3, rows0, rows1, rows2, rows3, zer,
             gsem0, gsem1, gsem2, gsem3, ssem0, ssem1, ssem2, ssem3,
             isem0, isem1, isem2, isem3):
    c = lax.axis_index("c")
    s = lax.axis_index("s")
    ebufs = [ebuf0, ebuf1, ebuf2, ebuf3]
    isems = [isem0, isem1, isem2, isem3]
    rowss = [rows0, rows1, rows2, rows3]
    gsems = [gsem0, gsem1, gsem2, gsem3]
    ssems = [ssem0, ssem1, ssem2, ssem3]
    coff = c * NP

    zf = jnp.zeros((16,), jnp.float32)

    def zfill(i, _):
        zer[i, pl.ds(0, 16)] = zf
        zer[i, pl.ds(16, 16)] = zf
        return 0

    lax.fori_loop(0, ZR, zfill, 0, unroll=4)

    base_r = s * RPT
    erow0 = s * ROWS_PT

    def idx_slice(kk):
        rn = jnp.minimum(kk, NCHUNK - 1)
        return eidx.at[pl.ds(erow0 + rn * K, K)]

    def fire_idx(kk, slot):
        pltpu.async_copy(idx_slice(kk), ebufs[slot], isems[slot])

    def wait_idx(kk, slot):
        pltpu.make_async_copy(idx_slice(kk), ebufs[slot], isems[slot]).wait()

    def adjust_src(slot):
        for g in range(IW // 16):
            ebufs[slot][0, 1, pl.ds(g * 16, 16)] = \
                ebufs[slot][0, 1, pl.ds(g * 16, 16)] + coff

    def one_layer(table, out, last=False):
        zcps = [pltpu.async_copy(zer, acc.at[pl.ds(base_r + z * ZR, ZR)],
                                 gsem0)
                for z in range(RPT // ZR)]
        for zc in zcps:
            zc.wait()
        plsc.subcore_barrier()

        def fire_g(m):
            pltpu.async_copy(table.at[ebufs[m].at[0, 1]],
                             rowss[m].at[0], gsems[m])

        def wait_g(m):
            pltpu.make_async_copy(table.at[ebufs[m].at[0, 1]],
                                  rowss[m].at[0], gsems[m]).wait()

        def fire_s(m):
            pltpu.async_copy(rowss[m].at[0], acc.at[ebufs[m].at[0, 0]],
                             ssems[m], add=True)

        def wait_s(m):
            pltpu.make_async_copy(rowss[m].at[0],
                                  acc.at[ebufs[m].at[0, 0]],
                                  ssems[m]).wait()

        def mul(m):
            def wmul(g, _):
                w16 = lax.bitcast_convert_type(
                    ebufs[m][0, 2, pl.ds(g * 16, 16)], jnp.float32)
                for i in range(16):
                    w = lax.gather(
                        w16, jnp.full((16, 1), i, jnp.int32),
                        dimension_numbers=lax.GatherDimensionNumbers(
                            offset_dims=(), collapsed_slice_dims=(0,),
                            start_index_map=(0,)),
                        slice_sizes=(1,),
                        mode=lax.GatherScatterMode.PROMISE_IN_BOUNDS)
                    r = g * 16 + i
                    rb = rowss[m]
                    rb[0, r, pl.ds(0, 16)] = rb[0, r, pl.ds(0, 16)] * w
                    rb[0, r, pl.ds(16, 16)] = rb[0, r, pl.ds(16, 16)] * w
                return 0

            lax.fori_loop(0, IW // 16, wmul, 0)

        fire_idx(0, 0)
        fire_idx(1, 1)
        wait_idx(0, 0)
        adjust_src(0)
        fire_g(0)

        @pl.loop(0, NCHUNK, step=4)
        def grp(k):
            for b in range(4):
                kk = k + b
                bn, bp = (b + 1) % 4, (b + 2) % 4

                def steady(b=b, bn=bn, bp=bp, kk=kk):
                    wait_s(bp)
                    fire_idx(kk + 2, bp)
                if b < 2:
                    pl.when(k > 0)(steady)
                    if b == 1:
                        @pl.when(k == 0)
                        def _(bp=bp, kk=kk):
                            fire_idx(kk + 2, bp)
                    else:
                        @pl.when(k == 0)
                        def _(bp=bp, kk=kk):
                            fire_idx(kk + 2, bp)
                else:
                    steady()
                wait_idx(kk + 1, bn)
                adjust_src(bn)
                fire_g(bn)
                wait_g(b)
                mul(b)
                fire_s(b)

        wait_s(2)
        wait_s(3)
        wait_g(0)
        wait_idx(NCHUNK + 1, 1)

        plsc.subcore_barrier()
        if not last:
            wcps = [pltpu.async_copy(
                        acc.at[pl.ds(base_r + z * ZR, ZR)],
                        out.at[pl.ds(c * NP + base_r + z * ZR, ZR)], gsem0)
                    for z in range(RPT // ZR)]
            for wc in wcps:
                wc.wait()
        else:
            MQ = 128
            third = jnp.float32(1.0 / NLAYERS)

            @pl.loop(0, RPT // MQ)
            def mq(qi):
                row = base_r + qi * MQ
                cp1 = pltpu.async_copy(l1.at[pl.ds(c * NP + row, MQ)],
                                       rows0.at[0], gsem0)
                cp2 = pltpu.async_copy(l2.at[pl.ds(c * NP + row, MQ)],
                                       rows1.at[0], gsem1)
                cp3 = pltpu.async_copy(acc.at[pl.ds(row, MQ)],
                                       rows2.at[0], gsem2)
                cp1.wait()
                cp2.wait()
                cp3.wait()

                def mbody(i, _):
                    for hh in (0, 16):
                        v = (rows0[0, i, pl.ds(hh, 16)]
                             + rows1[0, i, pl.ds(hh, 16)]
                             + rows2[0, i, pl.ds(hh, 16)])
                        rows3[0, i, pl.ds(hh, 16)] = v * third
                    return 0

                lax.fori_loop(0, MQ, mbody, 0, unroll=4)
                pltpu.sync_copy(rows3.at[0], out.at[pl.ds(c * NP + row, MQ)])
        plsc.subcore_barrier()

    one_layer(egoT, l1)
    one_layer(l1, l2)
    one_layer(l2, mout, last=True)


def _propagate(egoT, eidx):
    mesh = plsc.VectorSubcoreMesh(core_axis_name="c", subcore_axis_name="s")
    sds = jax.ShapeDtypeStruct((2 * NP, H), jnp.float32)
    return pl.kernel(
        _sc_body,
        out_type=(sds, sds, sds),
        mesh=mesh,
        scratch_types=[
            pltpu.VMEM_SHARED((NP, H), jnp.float32),
        ] + [pltpu.VMEM((K, 3, IW), jnp.int32)] * 4
          + [pltpu.VMEM((K, IW, H), jnp.float32)] * 4
          + [pltpu.VMEM((ZR, H), jnp.float32)]
          + [pltpu.SemaphoreType.DMA] * 12,
        compiler_params=pltpu.CompilerParams(use_tc_tiling_on_sc=False),
    )(egoT, eidx)


def _assemble_body(a, b, o):
    o[:, :H] = a[...]
    o[:, H:] = b[...]


def _assemble(mout):
    BR = 400
    return pl.pallas_call(
        _assemble_body,
        grid=(N // BR,),
        in_specs=[pl.BlockSpec((BR, H), lambda i: (i, 0)),
                  pl.BlockSpec((BR, H), lambda i: (i + NP // BR, 0))],
        out_specs=pl.BlockSpec((BR, D), lambda i: (i, 0)),
        out_shape=jax.ShapeDtypeStruct((N, D), jnp.float32),
    )(mout, mout)


@jax.jit
def kernel(user_emb, item_emb, user_prototypes, item_prototypes,
           edge_index, edge_weight):
    ego0 = jnp.concatenate([user_emb, item_emb], axis=0)
    egoT = (jnp.zeros((2 * NP, H), jnp.float32)
            .at[:N].set(ego0[:, :H])
            .at[NP:NP + N].set(ego0[:, H:]))

    pad = EP - E
    dst_p = jnp.concatenate([edge_index[0], jnp.zeros((pad,), jnp.int32)])
    src_p = jnp.concatenate([edge_index[1], jnp.zeros((pad,), jnp.int32)])
    w_p = jnp.concatenate([edge_weight, jnp.zeros((pad,), jnp.float32)])
    wbits = lax.bitcast_convert_type(w_p, jnp.int32)
    eidx = jnp.concatenate(
        [dst_p.reshape(RW, 1, IW), src_p.reshape(RW, 1, IW),
         wbits.reshape(RW, 1, IW)], axis=1)

    l1, l2, mout = _propagate(egoT, eidx)

    all_emb = _assemble(mout)
    return (all_emb[:U_NUM], all_emb[U_NUM:],
            user_prototypes, item_prototypes)


# speedup vs baseline: 1.1137x; 1.0676x over previous
import functools

import jax
import jax.numpy as jnp
from jax import lax
from jax.experimental import pallas as pl
from jax.experimental.pallas import tpu as pltpu
from jax.experimental.pallas import tpu_sc as plsc

U_NUM = 25000
I_NUM = 25000
N = U_NUM + I_NUM
E = 800000
D = 64
H = 32
NLAYERS = 3

NC = 2
NS = 16
IW = 128
RW = 6400
EP = RW * IW
ROWS_PT = RW // NS
K = 1
NCHUNK = ROWS_PT // K
NP = 51200
RPT = NP // NS
ZR = 160


def _sc_body(egoT, eidx, l1, l2, mout,
             acc, ebuf0, ebuf1, ebuf2, ebuf3, ebuf4, ebuf5, ebuf6, ebuf7,
             rows0, rows1, rows2, rows3, zer,
             gsem0, gsem1, gsem2, gsem3, ssem0, ssem1, ssem2, ssem3,
             isem0, isem1, isem2, isem3, isem4, isem5, isem6, isem7):
    c = lax.axis_index("c")
    s = lax.axis_index("s")
    ebufs = [ebuf0, ebuf1, ebuf2, ebuf3, ebuf4, ebuf5, ebuf6, ebuf7]
    isems = [isem0, isem1, isem2, isem3, isem4, isem5, isem6, isem7]
    rowss = [rows0, rows1, rows2, rows3]
    gsems = [gsem0, gsem1, gsem2, gsem3]
    ssems = [ssem0, ssem1, ssem2, ssem3]
    coff = c * NP

    zf = jnp.zeros((16,), jnp.float32)

    def zfill(i, _):
        zer[i, pl.ds(0, 16)] = zf
        zer[i, pl.ds(16, 16)] = zf
        return 0

    lax.fori_loop(0, ZR, zfill, 0, unroll=4)

    base_r = s * RPT
    erow0 = s * ROWS_PT

    def idx_slice(kk):
        rn = jnp.minimum(kk, NCHUNK - 1)
        return eidx.at[pl.ds(erow0 + rn * K, K)]

    def fire_idx(kk, slot):
        pltpu.async_copy(idx_slice(kk), ebufs[slot], isems[slot])

    def wait_idx(kk, slot):
        pltpu.make_async_copy(idx_slice(kk), ebufs[slot], isems[slot]).wait()

    def adjust_src(slot):
        for g in range(IW // 16):
            ebufs[slot][0, 1, pl.ds(g * 16, 16)] = \
                ebufs[slot][0, 1, pl.ds(g * 16, 16)] + coff

    def one_layer(table, out, last=False):
        zcps = [pltpu.async_copy(zer, acc.at[pl.ds(base_r + z * ZR, ZR)],
                                 gsem0)
                for z in range(RPT // ZR)]
        for zc in zcps:
            zc.wait()
        plsc.subcore_barrier()

        def fire_g(e, r):
            pltpu.async_copy(table.at[ebufs[e].at[0, 1]],
                             rowss[r].at[0], gsems[r])

        def wait_g(e, r):
            pltpu.make_async_copy(table.at[ebufs[e].at[0, 1]],
                                  rowss[r].at[0], gsems[r]).wait()

        def fire_s(e, r):
            pltpu.async_copy(rowss[r].at[0], acc.at[ebufs[e].at[0, 0]],
                             ssems[r], add=True)

        def wait_s(e, r):
            pltpu.make_async_copy(rowss[r].at[0],
                                  acc.at[ebufs[e].at[0, 0]],
                                  ssems[r]).wait()

        def mul(e, r):
            def wmul(g, _):
                w16 = lax.bitcast_convert_type(
                    ebufs[e][0, 2, pl.ds(g * 16, 16)], jnp.float32)
                for i in range(16):
                    w = lax.gather(
                        w16, jnp.full((16, 1), i, jnp.int32),
                        dimension_numbers=lax.GatherDimensionNumbers(
                            offset_dims=(), collapsed_slice_dims=(0,),
                            start_index_map=(0,)),
                        slice_sizes=(1,),
                        mode=lax.GatherScatterMode.PROMISE_IN_BOUNDS)
                    rr = g * 16 + i
                    rb = rowss[r]
                    rb[0, rr, pl.ds(0, 16)] = rb[0, rr, pl.ds(0, 16)] * w
                    rb[0, rr, pl.ds(16, 16)] = rb[0, rr, pl.ds(16, 16)] * w
                return 0

            lax.fori_loop(0, IW // 16, wmul, 0)

        for p in range(4):
            fire_idx(p, p)
        for p in range(2):
            wait_idx(p, p)
            adjust_src(p)
            fire_g(p, p)

        @pl.loop(0, NCHUNK, step=8)
        def grp(k):
            for b in range(8):
                kk = k + b

                def steady(b=b, kk=kk):
                    wait_s((b + 6) % 8, (b + 2) % 4)
                if b < 2:
                    pl.when(k > 0)(steady)
                else:
                    steady()
                fire_idx(kk + 4, (b + 4) % 8)
                wait_idx(kk + 2, (b + 2) % 8)
                adjust_src((b + 2) % 8)
                fire_g((b + 2) % 8, (b + 2) % 4)
                wait_g(b, b % 4)
                mul(b, b % 4)
                fire_s(b, b % 4)

        wait_s(6, 2)
        wait_s(7, 3)
        wait_g(0, 0)
        wait_g(1, 1)
        wait_idx(NCHUNK + 2, 2)
        wait_idx(NCHUNK + 3, 3)

        plsc.subcore_barrier()
        if not last:
            wcps = [pltpu.async_copy(
                        acc.at[pl.ds(base_r + z * ZR, ZR)],
                        out.at[pl.ds(c * NP + base_r + z * ZR, ZR)], gsem0)
                    for z in range(RPT // ZR)]
            for wc in wcps:
                wc.wait()
        else:
            MQ = 128
            third = jnp.float32(1.0 / NLAYERS)

            @pl.loop(0, RPT // MQ)
            def mq(qi):
                row = base_r + qi * MQ
                cp1 = pltpu.async_copy(l1.at[pl.ds(c * NP + row, MQ)],
                                       rows0.at[0], gsem0)
                cp2 = pltpu.async_copy(l2.at[pl.ds(c * NP + row, MQ)],
                                       rows1.at[0], gsem1)
                cp3 = pltpu.async_copy(acc.at[pl.ds(row, MQ)],
                                       rows2.at[0], gsem2)
                cp1.wait()
                cp2.wait()
                cp3.wait()

                def mbody(i, _):
                    for hh in (0, 16):
                        v = (rows0[0, i, pl.ds(hh, 16)]
                             + rows1[0, i, pl.ds(hh, 16)]
                             + rows2[0, i, pl.ds(hh, 16)])
                        rows3[0, i, pl.ds(hh, 16)] = v * third
                    return 0

                lax.fori_loop(0, MQ, mbody, 0, unroll=4)
                pltpu.sync_copy(rows3.at[0], out.at[pl.ds(c * NP + row, MQ)])
        plsc.subcore_barrier()

    one_layer(egoT, l1)
    one_layer(l1, l2)
    one_layer(l2, mout, last=True)


def _propagate(egoT, eidx):
    mesh = plsc.VectorSubcoreMesh(core_axis_name="c", subcore_axis_name="s")
    sds = jax.ShapeDtypeStruct((2 * NP, H), jnp.float32)
    return pl.kernel(
        _sc_body,
        out_type=(sds, sds, sds),
        mesh=mesh,
        scratch_types=[
            pltpu.VMEM_SHARED((NP, H), jnp.float32),
        ] + [pltpu.VMEM((K, 3, IW), jnp.int32)] * 8
          + [pltpu.VMEM((K, IW, H), jnp.float32)] * 4
          + [pltpu.VMEM((ZR, H), jnp.float32)]
          + [pltpu.SemaphoreType.DMA] * 16,
        compiler_params=pltpu.CompilerParams(use_tc_tiling_on_sc=False),
    )(egoT, eidx)


def _assemble_body(a, b, o):
    o[:, :H] = a[...]
    o[:, H:] = b[...]


def _assemble(mout):
    BR = 400
    return pl.pallas_call(
        _assemble_body,
        grid=(N // BR,),
        in_specs=[pl.BlockSpec((BR, H), lambda i: (i, 0)),
                  pl.BlockSpec((BR, H), lambda i: (i + NP // BR, 0))],
        out_specs=pl.BlockSpec((BR, D), lambda i: (i, 0)),
        out_shape=jax.ShapeDtypeStruct((N, D), jnp.float32),
    )(mout, mout)


@jax.jit
def kernel(user_emb, item_emb, user_prototypes, item_prototypes,
           edge_index, edge_weight):
    ego0 = jnp.concatenate([user_emb, item_emb], axis=0)
    egoT = (jnp.zeros((2 * NP, H), jnp.float32)
            .at[:N].set(ego0[:, :H])
            .at[NP:NP + N].set(ego0[:, H:]))

    pad = EP - E
    dst_p = jnp.concatenate([edge_index[0], jnp.zeros((pad,), jnp.int32)])
    src_p = jnp.concatenate([edge_index[1], jnp.zeros((pad,), jnp.int32)])
    w_p = jnp.concatenate([edge_weight, jnp.zeros((pad,), jnp.float32)])
    wbits = lax.bitcast_convert_type(w_p, jnp.int32)
    eidx = jnp.concatenate(
        [dst_p.reshape(RW, 1, IW), src_p.reshape(RW, 1, IW),
         wbits.reshape(RW, 1, IW)], axis=1)

    l1, l2, mout = _propagate(egoT, eidx)

    all_emb = _assemble(mout)
    return (all_emb[:U_NUM], all_emb[U_NUM:],
            user_prototypes, item_prototypes)
